# Initial kernel scaffold; baseline (speedup 1.0000x reference)
import jax
import jax.numpy as jnp
from jax.experimental import pallas as pl


def kernel(indices, table):
    raise NotImplementedError("write your pallas kernel here")


# speedup vs baseline: 1.0313x; 1.0313x over previous
import functools

import jax
import jax.numpy as jnp
from jax import lax
from jax.experimental import pallas as pl
from jax.experimental.pallas import tpu as pltpu
from jax.experimental.pallas import tpu_sc as plsc

_NUM_Q = 100000
_D = 200
_B = 4096
_L = 50
_N = _B * _L

_info = plsc.get_sparse_core_info()
_NC = _info.num_cores
_NS = _info.num_subcores
_NW = _NC * _NS
_CH = 128
_PER_W = _N // _NW
_STEPS = _PER_W // _CH

_mesh = plsc.VectorSubcoreMesh(core_axis_name="c", subcore_axis_name="s")


@functools.partial(
    pl.kernel,
    out_type=jax.ShapeDtypeStruct((_N, _D), jnp.float32),
    mesh=_mesh,
    scratch_types=[
        pltpu.VMEM((1, _STEPS, _CH), jnp.int32),
        pltpu.VMEM((_CH, _D), jnp.float32),
        pltpu.SemaphoreType.DMA,
    ],
    compiler_params=pltpu.CompilerParams(use_tc_tiling_on_sc=False),
)
def _gather(table_hbm, idx_hbm, out_hbm, idx_v, rows_v, gsem):
    wid = lax.axis_index("s") * _NC + lax.axis_index("c")
    base = wid * _PER_W
    pltpu.sync_copy(idx_hbm.at[pl.ds(wid, 1)], idx_v)

    def step(j, carry):
        pltpu.async_copy(table_hbm.at[idx_v.at[0, j]], rows_v, gsem).wait()
        pltpu.sync_copy(rows_v, out_hbm.at[pl.ds(base + j * _CH, _CH)])
        return carry

    lax.fori_loop(0, _STEPS, step, 0)


def kernel(indices, table):
    idx2 = indices.reshape(_NW, _STEPS, _CH)
    out = _gather(table, idx2)
    return out.reshape(_B, _L, _D)


# speedup vs baseline: 1.0715x; 1.0389x over previous
import functools

import jax
import jax.numpy as jnp
from jax import lax
from jax.experimental import pallas as pl
from jax.experimental.pallas import tpu as pltpu
from jax.experimental.pallas import tpu_sc as plsc

_NUM_Q = 100000
_D = 200
_B = 4096
_L = 50
_N = _B * _L

_info = plsc.get_sparse_core_info()
_NC = _info.num_cores
_NS = _info.num_subcores
_NW = _NC * _NS
_CH = 64
_NBUF = 4
_PER_W = _N // _NW
_STEPS = _PER_W // _CH
_GROUPS = _STEPS // _NBUF

_mesh = plsc.VectorSubcoreMesh(core_axis_name="c", subcore_axis_name="s")


@functools.partial(
    pl.kernel,
    out_type=jax.ShapeDtypeStruct((_N, _D), jnp.float32),
    mesh=_mesh,
    scratch_types=[
        pltpu.VMEM((1, _STEPS, _CH), jnp.int32),
        pltpu.VMEM((_CH, _D), jnp.float32),
        pltpu.VMEM((_CH, _D), jnp.float32),
        pltpu.VMEM((_CH, _D), jnp.float32),
        pltpu.VMEM((_CH, _D), jnp.float32),
        pltpu.SemaphoreType.DMA,
        pltpu.SemaphoreType.DMA,
        pltpu.SemaphoreType.DMA,
        pltpu.SemaphoreType.DMA,
        pltpu.SemaphoreType.DMA,
        pltpu.SemaphoreType.DMA,
        pltpu.SemaphoreType.DMA,
        pltpu.SemaphoreType.DMA,
    ],
    compiler_params=pltpu.CompilerParams(use_tc_tiling_on_sc=False),
)
def _gather(table_hbm, idx_hbm, out_hbm, idx_v, r0, r1, r2, r3,
            g0, g1, g2, g3, s0, s1, s2, s3):
    rows = (r0, r1, r2, r3)
    gsems = (g0, g1, g2, g3)
    ssems = (s0, s1, s2, s3)
    wid = lax.axis_index("s") * _NC + lax.axis_index("c")
    base = wid * _PER_W
    pltpu.sync_copy(idx_hbm.at[pl.ds(wid, 1)], idx_v)

    def start_gather(j, b):
        pltpu.async_copy(table_hbm.at[idx_v.at[0, j]], rows[b], gsems[b])

    def wait_gather(b):
        pltpu.make_async_copy(table_hbm.at[pl.ds(0, _CH)], rows[b],
                              gsems[b]).wait()

    def start_store(j, b):
        pltpu.async_copy(rows[b], out_hbm.at[pl.ds(base + j * _CH, _CH)],
                         ssems[b])

    def wait_store(b):
        pltpu.make_async_copy(rows[b], out_hbm.at[pl.ds(base, _CH)],
                              ssems[b]).wait()

    for b in range(_NBUF):
        start_gather(b, b)

    def group(g, carry):
        j0 = g * _NBUF
        for b in range(_NBUF):
            wait_gather(b)
            start_store(j0 + b, b)

            @pl.when(g + 1 < _GROUPS)
            def _():
                wait_store(b)
                start_gather(j0 + b + _NBUF, b)
        return carry

    lax.fori_loop(0, _GROUPS, group, 0)
    for b in range(_NBUF):
        wait_store(b)


def kernel(indices, table):
    idx2 = indices.reshape(_NW, _STEPS, _CH)
    out = _gather(table, idx2)
    return out.reshape(_B, _L, _D)
